# baseline (speedup 1.0000x reference)
import functools

import jax
import jax.numpy as jnp
from jax import lax
from jax.experimental import pallas as pl
from jax.experimental.pallas import tpu as pltpu
from jax.experimental.pallas import tpu_sc as plsc

_B = 2
_V = 4
_P = 2048
_S = 49152
_C = 64
_NVAR = 8

_NW = 32
_NSLOT = _B * _V
_LANES = 16
_UNROLL = 32
_GRPS = _P // _LANES
_PAD = 127


def _make_sc_gather():
    info = plsc.get_sparse_core_info()
    nc = info.num_cores

    mesh = plsc.VectorSubcoreMesh(core_axis_name="c", subcore_axis_name="s")

    @functools.partial(
        pl.kernel,
        mesh=mesh,
        compiler_params=pltpu.CompilerParams(needs_layout_passes=False),
        out_type=jax.ShapeDtypeStruct((_NSLOT * _C, _P), jnp.float32),
        scratch_types=[
            pltpu.VMEM((_B, _P), jnp.int32),
            pltpu.VMEM((_LANES,), jnp.int32),
            pltpu.VMEM((2, _S), jnp.float32),
            pltpu.VMEM((2, _P), jnp.float32),
            pltpu.SemaphoreType.DMA,
            pltpu.SemaphoreType.DMA,
        ],
    )
    def gather_kernel(varp_hbm, patch_hbm, table_hbm, out_hbm,
                      patch_v, varp_v, row_v, out_v, sem_in, sem_out):
        wid = lax.axis_index("s") * nc + lax.axis_index("c")
        pltpu.sync_copy(patch_hbm, patch_v)
        pltpu.sync_copy(varp_hbm, varp_v)
        lanes = lax.iota(jnp.int32, _LANES)
        var_vec = varp_v[...]

        first = lanes
        for t in range(_NSLOT - 1, -1, -1):
            vt = lax.reduce_max(
                jnp.where(lanes == t, var_vec, -1), axes=(0,))
            first = jnp.where(var_vec == vt, t, first)
        is_leader = jnp.logical_and(first == lanes, lanes < _NSLOT)
        pos = plsc.cumsum(jnp.where(is_leader, 1, 0).astype(jnp.int32)) - 1
        d_cnt = lax.reduce_max(pos, axes=(0,)) + 1
        nrows = (d_cnt * _C) // _NW

        def rowid(k):
            r = wid + _NW * k
            d = r // _C
            c = lax.rem(r, _C)
            sel = jnp.logical_and(pos == d, is_leader)
            uvar = lax.reduce_max(
                jnp.where(sel, var_vec, -1), axes=(0,))
            return d, c, uvar

        def start_row(k, buf):
            _, c, uvar = rowid(k)
            pltpu.make_async_copy(
                table_hbm.at[pl.ds(uvar * _C + c, 1)],
                row_v.at[pl.ds(buf, 1)], sem_in).start()

        start_row(0, 0)
        start_row(1, 1)

        def row_body(k, m):
            d, c, uvar = rowid(k)
            buf = lax.rem(k, 2)
            bufv = jnp.full((_LANES,), buf, jnp.int32)
            pltpu.make_async_copy(
                table_hbm.at[pl.ds(0, 1)], row_v.at[pl.ds(buf, 1)],
                sem_in).wait()
            match = var_vec == uvar
            match_i = jnp.where(match, 1, 0).astype(jnp.int32)
            n_d = jnp.sum(match_i)
            rank = plsc.cumsum(match_i) - 1

            def slot_body(j, m2):
                slot = lax.reduce_max(
                    jnp.where(jnp.logical_and(match, rank == j), lanes, -1),
                    axes=(0,))
                b_j = slot // _V
                par = lax.rem(m2, 2)
                parv = jnp.full((_LANES,), par, jnp.int32)

                @pl.when(m2 >= 2)
                def _():
                    pltpu.make_async_copy(
                        out_v.at[pl.ds(par, 1)], out_hbm.at[pl.ds(0, 1)],
                        sem_out).wait()

                @plsc.parallel_loop(0, _GRPS, step=1, unroll=_UNROLL)
                def _gather(g):
                    o = g * _LANES
                    idx = patch_v[b_j, pl.ds(o, _LANES)]
                    out_v[par, pl.ds(o, _LANES)] = plsc.load_gather(
                        row_v, [bufv, idx])
                pltpu.make_async_copy(
                    out_v.at[pl.ds(par, 1)],
                    out_hbm.at[pl.ds(slot * _C + c, 1)], sem_out).start()
                return m2 + 1

            m = lax.fori_loop(0, n_d, slot_body, m)

            @pl.when(k + 2 < nrows)
            def _():
                start_row(k + 2, buf)

            return m

        m = lax.fori_loop(0, nrows, row_body, 0)
        pltpu.make_async_copy(
            out_v.at[pl.ds(0, 1)], out_hbm.at[pl.ds(0, 1)], sem_out).wait()
        pltpu.make_async_copy(
            out_v.at[pl.ds(0, 1)], out_hbm.at[pl.ds(0, 1)], sem_out).wait()

    return gather_kernel


def kernel(var_indices, patch_idx, mg_embedding):
    table = jnp.transpose(mg_embedding, (0, 2, 1)).reshape(_NVAR * _C, _S)
    var_flat = var_indices.reshape(-1).astype(jnp.int32)
    varp = jnp.concatenate(
        [var_flat, jnp.full((_LANES - _NSLOT,), _PAD, jnp.int32)])
    patch2 = patch_idx.reshape(_B, _P).astype(jnp.int32)
    out = _make_sc_gather()(varp, patch2, table)
    return jnp.transpose(
        out.reshape(_B, _V, _C, _P), (0, 1, 3, 2))[:, :, None, :, :]


# speedup vs baseline: 1.0683x; 1.0683x over previous
import functools

import jax
import jax.numpy as jnp
from jax import lax
from jax.experimental import pallas as pl
from jax.experimental.pallas import tpu as pltpu
from jax.experimental.pallas import tpu_sc as plsc

_B = 2
_V = 4
_P = 2048
_S = 49152
_C = 64
_NVAR = 8

_NW = 32
_NSLOT = _B * _V
_LANES = 16
_UNROLL = 16
_GRPS = _P // _LANES
_PAD = 127


def _make_sc_gather():
    info = plsc.get_sparse_core_info()
    nc = info.num_cores

    mesh = plsc.VectorSubcoreMesh(core_axis_name="c", subcore_axis_name="s")

    @functools.partial(
        pl.kernel,
        mesh=mesh,
        compiler_params=pltpu.CompilerParams(needs_layout_passes=False),
        out_type=jax.ShapeDtypeStruct((_NSLOT * _C, _P), jnp.float32),
        scratch_types=[
            pltpu.VMEM((_B, _P), jnp.int32),
            pltpu.VMEM((_LANES,), jnp.int32),
            pltpu.VMEM((2, _S), jnp.float32),
            pltpu.VMEM((2, _P), jnp.float32),
            pltpu.SemaphoreType.DMA,
            pltpu.SemaphoreType.DMA,
        ],
    )
    def gather_kernel(varp_hbm, patch_hbm, table_hbm, out_hbm,
                      patch_v, varp_v, row_v, out_v, sem_in, sem_out):
        wid = lax.axis_index("s") * nc + lax.axis_index("c")
        pltpu.sync_copy(patch_hbm, patch_v)
        pltpu.sync_copy(varp_hbm, varp_v)
        lanes = lax.iota(jnp.int32, _LANES)
        var_vec = varp_v[...]

        first = lanes
        for t in range(_NSLOT - 1, -1, -1):
            vt = lax.reduce_max(
                jnp.where(lanes == t, var_vec, -1), axes=(0,))
            first = jnp.where(var_vec == vt, t, first)
        is_leader = jnp.logical_and(first == lanes, lanes < _NSLOT)
        pos = plsc.cumsum(jnp.where(is_leader, 1, 0).astype(jnp.int32)) - 1
        d_cnt = lax.reduce_max(pos, axes=(0,)) + 1
        nrows = (d_cnt * _C) // _NW

        def rowid(k):
            r = wid + _NW * k
            d = r // _C
            c = lax.rem(r, _C)
            sel = jnp.logical_and(pos == d, is_leader)
            uvar = lax.reduce_max(
                jnp.where(sel, var_vec, -1), axes=(0,))
            return d, c, uvar

        def start_row(k, buf):
            _, c, uvar = rowid(k)
            pltpu.make_async_copy(
                table_hbm.at[pl.ds(uvar * _C + c, 1)],
                row_v.at[pl.ds(buf, 1)], sem_in).start()

        start_row(0, 0)
        start_row(1, 1)

        lane_b = lanes // _V

        def drain(n):
            def w_body(i, cw):
                pltpu.make_async_copy(
                    out_v.at[pl.ds(0, 1)], out_hbm.at[pl.ds(0, 1)],
                    sem_out).wait()
                return cw

            lax.fori_loop(0, n, w_body, 0)

        def row_body(k, carry):
            g_cnt, p0, p1 = carry
            d, c, uvar = rowid(k)
            buf = lax.rem(k, 2)
            bufv = jnp.full((_LANES,), buf, jnp.int32)
            pltpu.make_async_copy(
                table_hbm.at[pl.ds(0, 1)], row_v.at[pl.ds(buf, 1)],
                sem_in).wait()
            match = var_vec == uvar
            match_i = jnp.where(match, 1, 0).astype(jnp.int32)
            n0 = jnp.sum(match_i * jnp.where(lane_b == 0, 1, 0))
            n1 = jnp.sum(match_i) - n0
            n_og = jnp.where(n0 > 0, 1, 0) + jnp.where(n1 > 0, 1, 0)

            def og_body(j2, carry2):
                g2, q0, q1 = carry2
                first_is0 = jnp.logical_and(j2 == 0, n0 > 0)
                bb = jnp.where(first_is0, 0, 1)
                n_b = jnp.where(first_is0, n0, n1)
                par = lax.rem(g2, 2)
                pend = jnp.where(par == 0, q0, q1)
                drain(pend)
                mask2_i = match_i * jnp.where(lane_b == bb, 1, 0)
                rank2 = plsc.cumsum(mask2_i) - 1

                @plsc.parallel_loop(0, _GRPS, step=1, unroll=_UNROLL)
                def _gather(g):
                    o = g * _LANES
                    idx = patch_v[bb, pl.ds(o, _LANES)]
                    out_v[par, pl.ds(o, _LANES)] = plsc.load_gather(
                        row_v, [bufv, idx])

                def cp_body(jj, cw):
                    slot = lax.reduce_max(
                        jnp.where(
                            jnp.logical_and(mask2_i > 0, rank2 == jj),
                            lanes, -1), axes=(0,))
                    pltpu.make_async_copy(
                        out_v.at[pl.ds(par, 1)],
                        out_hbm.at[pl.ds(slot * _C + c, 1)],
                        sem_out).start()
                    return cw

                lax.fori_loop(0, n_b, cp_body, 0)
                q0 = jnp.where(par == 0, n_b, q0)
                q1 = jnp.where(par == 1, n_b, q1)
                return g2 + 1, q0, q1

            g_cnt, p0, p1 = lax.fori_loop(
                0, n_og, og_body, (g_cnt, p0, p1))

            @pl.when(k + 2 < nrows)
            def _():
                start_row(k + 2, buf)

            return g_cnt, p0, p1

        _, p0, p1 = lax.fori_loop(0, nrows, row_body, (0, 0, 0))
        drain(p0 + p1)

    return gather_kernel


def kernel(var_indices, patch_idx, mg_embedding):
    table = jnp.transpose(mg_embedding, (0, 2, 1)).reshape(_NVAR * _C, _S)
    var_flat = var_indices.reshape(-1).astype(jnp.int32)
    varp = jnp.concatenate(
        [var_flat, jnp.full((_LANES - _NSLOT,), _PAD, jnp.int32)])
    patch2 = patch_idx.reshape(_B, _P).astype(jnp.int32)
    out = _make_sc_gather()(varp, patch2, table)
    return jnp.transpose(
        out.reshape(_B, _V, _C, _P), (0, 1, 3, 2))[:, :, None, :, :]
